# baseline (speedup 1.0000x reference)
import functools

import jax
import jax.numpy as jnp
from jax import lax
from jax.experimental import pallas as pl
from jax.experimental.pallas import tpu as pltpu
from jax.experimental.pallas import tpu_sc as plsc

N = 10000
E = 320000
F = 128
C = 64
K = 10

NC = 2
NS = 16
NW = NC * NS
EPW = E // NW
CHUNK = 64
EPW_PAD = 10048
NCHUNK = EPW_PAD // CHUNK
NP = N + 8
RPW = 624
REM = N - NS * RPW

_mesh = functools.partial(
    plsc.VectorSubcoreMesh, core_axis_name="c", subcore_axis_name="s")


@functools.partial(
    pl.kernel,
    out_type=jax.ShapeDtypeStruct((NW, N), jnp.float32),
    mesh=_mesh(),
    scratch_types=[
        pltpu.VMEM((EPW,), jnp.int32),
        pltpu.VMEM((N,), jnp.float32),
    ],
    compiler_params=pltpu.CompilerParams(needs_layout_passes=False),
)
def _deg_kernel(dst_hbm, out_hbm, idx_v, deg_v):
    cid = lax.axis_index("c")
    sid = lax.axis_index("s")
    wid = sid * NC + cid
    pltpu.sync_copy(dst_hbm.at[pl.ds(wid * EPW, EPW)], idx_v)

    def zero_body(i, carry):
        deg_v[pl.ds(i * 16, 16)] = jnp.zeros((16,), jnp.float32)
        return carry

    lax.fori_loop(0, N // 16, zero_body, 0)

    ones = jnp.ones((16,), jnp.float32)

    def acc_body(i, carry):
        d = idx_v[pl.ds(i * 16, 16)]
        plsc.addupdate_scatter(deg_v, [d], ones)
        return carry

    lax.fori_loop(0, EPW // 16, acc_body, 0)
    pltpu.sync_copy(deg_v, out_hbm.at[wid])


@functools.partial(
    pl.kernel,
    out_type=jax.ShapeDtypeStruct((NC, N, F), jnp.float32),
    mesh=_mesh(),
    scratch_types=[
        pltpu.VMEM((NCHUNK, CHUNK), jnp.int32),
        pltpu.VMEM((NCHUNK, CHUNK), jnp.int32),
        pltpu.VMEM((CHUNK, F), jnp.float32),
        pltpu.VMEM_SHARED((NP, F), jnp.float32),
        pltpu.SemaphoreType.DMA,
        pltpu.SemaphoreType.DMA,
        pltpu.SemaphoreType.DMA,
    ],
)
def _prop_kernel(hs_hbm, src_hbm, dst_hbm, zeros_hbm, out_hbm,
                 src_v, dst_v, rows_v, agg_sh, sem_src, sem_dst, sem_g):
    cid = lax.axis_index("c")
    sid = lax.axis_index("s")
    wid = sid * NC + cid

    pltpu.async_copy(src_hbm.at[wid], src_v, sem_src)
    pltpu.async_copy(dst_hbm.at[wid], dst_v, sem_dst)

    pltpu.sync_copy(zeros_hbm.at[pl.ds(sid * RPW, RPW)],
                    agg_sh.at[pl.ds(sid * RPW, RPW)])

    @pl.when(sid == NS - 1)
    def _():
        pltpu.sync_copy(zeros_hbm.at[pl.ds(NS * RPW, REM + 8)],
                        agg_sh.at[pl.ds(NS * RPW, REM + 8)])

    pltpu.make_async_copy(src_hbm.at[wid], src_v, sem_src).wait()
    pltpu.make_async_copy(dst_hbm.at[wid], dst_v, sem_dst).wait()
    plsc.subcore_barrier()

    def body(c, carry):
        pltpu.async_copy(
            hs_hbm.at[src_v.at[c]], rows_v, sem_g).wait()
        pltpu.sync_copy(rows_v, agg_sh.at[dst_v.at[c]], add=True)
        return carry

    lax.fori_loop(0, NCHUNK, body, 0)
    plsc.subcore_barrier()
    pltpu.sync_copy(agg_sh.at[pl.ds(sid * RPW, RPW)],
                    out_hbm.at[cid, pl.ds(sid * RPW, RPW)])

    @pl.when(sid == NS - 1)
    def _():
        pltpu.sync_copy(agg_sh.at[pl.ds(NS * RPW, REM)],
                        out_hbm.at[cid, pl.ds(NS * RPW, REM)])


def _tc_pre_body(feat_ref, w0_ref, b0_ref, wy_ref, wg_ref, bg_ref, degp_ref,
                 h0_ref, hs_ref, alpha_ref, norm_ref):
    feat = feat_ref[...]
    h0 = jnp.dot(feat, w0_ref[...], preferred_element_type=jnp.float32)
    h0 = jnp.maximum(h0 + b0_ref[...], 0.0)
    ylog = jnp.dot(feat, wy_ref[...], preferred_element_type=jnp.float32)
    ylog = ylog - jnp.max(ylog, axis=1, keepdims=True)
    e = jnp.exp(ylog)
    logits = e / jnp.sum(e, axis=1, keepdims=True)
    gate = jnp.dot(logits, wg_ref[...], preferred_element_type=jnp.float32)
    alpha = jax.nn.sigmoid(gate + bg_ref[...])
    deg = jnp.sum(degp_ref[...], axis=0, keepdims=True)
    norm = lax.rsqrt(jnp.maximum(deg, 1.0)).reshape(N, 1)
    h0_ref[...] = h0
    hs_ref[pl.ds(0, N), :] = h0 * norm
    hs_ref[pl.ds(N, 8), :] = jnp.zeros((8, F), jnp.float32)
    alpha_ref[...] = alpha
    norm_ref[...] = norm


_tc_pre = pl.pallas_call(
    _tc_pre_body,
    out_shape=[
        jax.ShapeDtypeStruct((N, F), jnp.float32),
        jax.ShapeDtypeStruct((NP, F), jnp.float32),
        jax.ShapeDtypeStruct((N, 1), jnp.float32),
        jax.ShapeDtypeStruct((N, 1), jnp.float32),
    ],
)


def _tc_upd_body(parts_ref, alpha_ref, norm_ref, h0_ref, h_ref, hs_ref):
    norm = norm_ref[...]
    alpha = alpha_ref[...]
    agg = (parts_ref[0] + parts_ref[1]) * norm
    h = (1.0 - alpha) * agg + alpha * h0_ref[...]
    h_ref[...] = h
    hs_ref[pl.ds(0, N), :] = h * norm
    hs_ref[pl.ds(N, 8), :] = jnp.zeros((8, F), jnp.float32)


_tc_upd = pl.pallas_call(
    _tc_upd_body,
    out_shape=[
        jax.ShapeDtypeStruct((N, F), jnp.float32),
        jax.ShapeDtypeStruct((NP, F), jnp.float32),
    ],
)


def kernel(features, edge_index, W0, b0, Wy, W_gate, b_gate):
    src = edge_index[0].reshape(NW, EPW)
    dst = edge_index[1].reshape(NW, EPW)
    pad = jnp.full((NW, EPW_PAD - EPW), N, jnp.int32)
    srcp = jnp.concatenate([src, pad], axis=1).reshape(NW, NCHUNK, CHUNK)
    dstp = jnp.concatenate([dst, pad], axis=1).reshape(NW, NCHUNK, CHUNK)

    deg_parts = _deg_kernel(edge_index[1])
    h0, hs, alpha, norm = _tc_pre(
        features, W0, b0.reshape(1, F), Wy, W_gate, b_gate.reshape(1, 1),
        deg_parts)

    zeros = jnp.zeros((NP, F), jnp.float32)
    h = h0
    for _ in range(K):
        parts = _prop_kernel(hs, srcp, dstp, zeros)
        h, hs = _tc_upd(parts, alpha, norm, h0)
    return h


# speedup vs baseline: 1.3103x; 1.3103x over previous
import functools

import jax
import jax.numpy as jnp
from jax import lax
from jax.experimental import pallas as pl
from jax.experimental.pallas import tpu as pltpu
from jax.experimental.pallas import tpu_sc as plsc

N = 10000
E = 320000
F = 128
C = 64
K = 10

NC = 2
NS = 16
NW = NC * NS
EPW = E // NW
CHUNK = 80
EPW_PAD = 10000
NCHUNK = EPW_PAD // CHUNK
NP = N + 8
RPW = 624
REM = N - NS * RPW

_mesh = functools.partial(
    plsc.VectorSubcoreMesh, core_axis_name="c", subcore_axis_name="s")


@functools.partial(
    pl.kernel,
    out_type=jax.ShapeDtypeStruct((NW, N), jnp.float32),
    mesh=_mesh(),
    scratch_types=[
        pltpu.VMEM((EPW,), jnp.int32),
        pltpu.VMEM((N,), jnp.float32),
    ],
    compiler_params=pltpu.CompilerParams(needs_layout_passes=False),
)
def _deg_kernel(dst_hbm, out_hbm, idx_v, deg_v):
    cid = lax.axis_index("c")
    sid = lax.axis_index("s")
    wid = sid * NC + cid
    pltpu.sync_copy(dst_hbm.at[pl.ds(wid * EPW, EPW)], idx_v)

    def zero_body(i, carry):
        deg_v[pl.ds(i * 16, 16)] = jnp.zeros((16,), jnp.float32)
        return carry

    lax.fori_loop(0, N // 16, zero_body, 0)

    ones = jnp.ones((16,), jnp.float32)

    def acc_body(i, carry):
        d = idx_v[pl.ds(i * 16, 16)]
        plsc.addupdate_scatter(deg_v, [d], ones)
        return carry

    lax.fori_loop(0, EPW // 16, acc_body, 0)
    pltpu.sync_copy(deg_v, out_hbm.at[wid])


@functools.partial(
    pl.kernel,
    out_type=jax.ShapeDtypeStruct((NC, N, F), jnp.float32),
    mesh=_mesh(),
    scratch_types=[
        pltpu.VMEM((NCHUNK, CHUNK), jnp.int32),
        pltpu.VMEM((NCHUNK, CHUNK), jnp.int32),
        pltpu.VMEM((CHUNK, F), jnp.float32),
        pltpu.VMEM_SHARED((NP, F), jnp.float32),
        pltpu.SemaphoreType.DMA,
        pltpu.SemaphoreType.DMA,
        pltpu.SemaphoreType.DMA,
    ],
)
def _prop_kernel(hs_hbm, src_hbm, dst_hbm, zeros_hbm, out_hbm,
                 src_v, dst_v, rows_v, agg_sh, sem_src, sem_dst, sem_g):
    cid = lax.axis_index("c")
    sid = lax.axis_index("s")
    wid = sid * NC + cid

    pltpu.async_copy(src_hbm.at[wid], src_v, sem_src)
    pltpu.async_copy(dst_hbm.at[wid], dst_v, sem_dst)

    pltpu.sync_copy(zeros_hbm.at[pl.ds(sid * RPW, RPW)],
                    agg_sh.at[pl.ds(sid * RPW, RPW)])

    @pl.when(sid == NS - 1)
    def _():
        pltpu.sync_copy(zeros_hbm.at[pl.ds(NS * RPW, REM + 8)],
                        agg_sh.at[pl.ds(NS * RPW, REM + 8)])

    pltpu.make_async_copy(src_hbm.at[wid], src_v, sem_src).wait()
    pltpu.make_async_copy(dst_hbm.at[wid], dst_v, sem_dst).wait()
    plsc.subcore_barrier()

    def body(c, carry):
        pltpu.async_copy(
            hs_hbm.at[src_v.at[c]], rows_v, sem_g).wait()
        pltpu.sync_copy(rows_v, agg_sh.at[dst_v.at[c]], add=True)
        return carry

    lax.fori_loop(0, NCHUNK, body, 0)
    plsc.subcore_barrier()
    pltpu.sync_copy(agg_sh.at[pl.ds(sid * RPW, RPW)],
                    out_hbm.at[cid, pl.ds(sid * RPW, RPW)])

    @pl.when(sid == NS - 1)
    def _():
        pltpu.sync_copy(agg_sh.at[pl.ds(NS * RPW, REM)],
                        out_hbm.at[cid, pl.ds(NS * RPW, REM)])


def _tc_pre_body(feat_ref, w0_ref, b0_ref, wy_ref, wg_ref, bg_ref, degp_ref,
                 h0_ref, hs_ref, alpha_ref, norm_ref):
    feat = feat_ref[...]
    h0 = jnp.dot(feat, w0_ref[...], preferred_element_type=jnp.float32)
    h0 = jnp.maximum(h0 + b0_ref[...], 0.0)
    ylog = jnp.dot(feat, wy_ref[...], preferred_element_type=jnp.float32)
    ylog = ylog - jnp.max(ylog, axis=1, keepdims=True)
    e = jnp.exp(ylog)
    logits = e / jnp.sum(e, axis=1, keepdims=True)
    gate = jnp.dot(logits, wg_ref[...], preferred_element_type=jnp.float32)
    alpha = jax.nn.sigmoid(gate + bg_ref[...])
    deg = jnp.sum(degp_ref[...], axis=0, keepdims=True)
    norm = lax.rsqrt(jnp.maximum(deg, 1.0)).reshape(N, 1)
    h0_ref[...] = h0
    hs_ref[pl.ds(0, N), :] = h0 * norm
    hs_ref[pl.ds(N, 8), :] = jnp.zeros((8, F), jnp.float32)
    alpha_ref[...] = alpha
    norm_ref[...] = norm


_tc_pre = pl.pallas_call(
    _tc_pre_body,
    out_shape=[
        jax.ShapeDtypeStruct((N, F), jnp.float32),
        jax.ShapeDtypeStruct((NP, F), jnp.float32),
        jax.ShapeDtypeStruct((N, 1), jnp.float32),
        jax.ShapeDtypeStruct((N, 1), jnp.float32),
    ],
)


def _tc_upd_body(parts_ref, alpha_ref, norm_ref, h0_ref, h_ref, hs_ref):
    norm = norm_ref[...]
    alpha = alpha_ref[...]
    agg = (parts_ref[0] + parts_ref[1]) * norm
    h = (1.0 - alpha) * agg + alpha * h0_ref[...]
    h_ref[...] = h
    hs_ref[pl.ds(0, N), :] = h * norm
    hs_ref[pl.ds(N, 8), :] = jnp.zeros((8, F), jnp.float32)


_tc_upd = pl.pallas_call(
    _tc_upd_body,
    out_shape=[
        jax.ShapeDtypeStruct((N, F), jnp.float32),
        jax.ShapeDtypeStruct((NP, F), jnp.float32),
    ],
)


def kernel(features, edge_index, W0, b0, Wy, W_gate, b_gate):
    src = edge_index[0].reshape(NW, EPW)
    dst = edge_index[1].reshape(NW, EPW)
    pad = jnp.full((NW, EPW_PAD - EPW), N, jnp.int32)
    srcp = jnp.concatenate([src, pad], axis=1).reshape(NW, NCHUNK, CHUNK)
    dstp = jnp.concatenate([dst, pad], axis=1).reshape(NW, NCHUNK, CHUNK)

    deg_parts = _deg_kernel(edge_index[1])
    h0, hs, alpha, norm = _tc_pre(
        features, W0, b0.reshape(1, F), Wy, W_gate, b_gate.reshape(1, 1),
        deg_parts)

    zeros = jnp.zeros((NP, F), jnp.float32)
    h = h0
    for _ in range(K):
        parts = _prop_kernel(hs, srcp, dstp, zeros)
        h, hs = _tc_upd(parts, alpha, norm, h0)
    return h
